# baseline (speedup 1.0000x reference)
import functools

import jax
import jax.numpy as jnp
from jax import lax
from jax.experimental import pallas as pl
from jax.experimental.pallas import tpu as pltpu
from jax.experimental.pallas import tpu_sc as plsc

B, S, D = 4, 8192, 768
NC, NS = 2, 16
NW = NC * NS
S_PER_W = S // NW
CHUNK = 32
N_CHUNK = S_PER_W // CHUNK
CW = CHUNK * D


def _sc_body(x_hbm, t_hbm, o_hbm, x_v, t_v):
    wid = lax.axis_index("s") * NC + lax.axis_index("c")
    row_base = wid * S_PER_W

    def chunk_body(ci, carry):
        t_off = (row_base + ci * CHUNK) * D
        pltpu.sync_copy(t_hbm.at[pl.ds(t_off, CW)], t_v)

        def batch_body(b, carry2):
            x_off = b * (S * D) + t_off
            pltpu.sync_copy(x_hbm.at[pl.ds(x_off, CW)], x_v)

            def add_body(i, c3):
                base = i * 64
                for u in range(4):
                    off = base + u * 16
                    x_v[pl.ds(off, 16)] = x_v[pl.ds(off, 16)] + t_v[pl.ds(off, 16)]
                return c3

            lax.fori_loop(0, CW // 64, add_body, 0)
            pltpu.sync_copy(x_v, o_hbm.at[pl.ds(x_off, CW)])
            return carry2

        lax.fori_loop(0, B, batch_body, 0)
        return carry

    lax.fori_loop(0, N_CHUNK, chunk_body, 0)


@functools.partial(jax.jit)
def _sc_add(x_flat, t_flat):
    mesh = plsc.VectorSubcoreMesh(core_axis_name="c", subcore_axis_name="s")
    return pl.kernel(
        _sc_body,
        mesh=mesh,
        out_type=jax.ShapeDtypeStruct((B * S * D,), jnp.float32),
        scratch_types=[
            pltpu.VMEM((CW,), jnp.float32),
            pltpu.VMEM((CW,), jnp.float32),
        ],
    )(x_flat, t_flat)


def kernel(inputs, table):
    out = _sc_add(inputs.reshape(-1), table.reshape(-1))
    return out.reshape(B, S, D)
